# baseline (speedup 1.0000x reference)
import numpy as np
import jax
import jax.numpy as jnp
from jax.experimental import pallas as pl
from jax.experimental.pallas import tpu as pltpu

_N_BINS = 15
_B = np.linspace(0.0, 1.0, _N_BINS + 1)
_B_LO = np.zeros((16,), np.float32)
_B_HI = np.zeros((16,), np.float32)
_B_LO[:_N_BINS] = _B[:_N_BINS].astype(np.float32)
_B_HI[:_N_BINS] = _B[1:].astype(np.float32)
_B_LO[_N_BINS] = 2.0
_B_HI[_N_BINS] = 3.0


def _ece_body(logits_ref, labels_ref, bounds_ref, out_ref, acc_ref, *, n_total):
    j = pl.program_id(0)
    nsteps = pl.num_programs(0)

    @pl.when(j == 0)
    def _init():
        acc_ref[...] = jnp.zeros_like(acc_ref)

    x = logits_ref[...]
    m = jnp.max(x, axis=1, keepdims=True)
    s = jnp.sum(jnp.exp(x - m), axis=1, keepdims=True)
    conf = 1.0 / s

    col = jax.lax.broadcasted_iota(jnp.int32, x.shape, 1)
    amax = jnp.min(jnp.where(x == m, col, x.shape[1]), axis=1, keepdims=True)
    lab = labels_ref[0]
    acc = (amax == lab).astype(jnp.float32)

    lo = bounds_ref[0:1, :]
    hi = bounds_ref[1:2, :]
    inb = ((conf > lo) & (conf <= hi)).astype(jnp.float32)
    acc_ref[0:1, 0:16] += jnp.sum(inb, axis=0, keepdims=True)
    acc_ref[1:2, 0:16] += jnp.sum(inb * conf, axis=0, keepdims=True)
    acc_ref[2:3, 0:16] += jnp.sum(inb * acc, axis=0, keepdims=True)

    @pl.when(j == nsteps - 1)
    def _fini():
        cnt = acc_ref[0:1, 0:16]
        csum = acc_ref[1:2, 0:16]
        asum = acc_ref[2:3, 0:16]
        safe = jnp.maximum(cnt, 1.0)
        contrib = jnp.abs(csum / safe - asum / safe) * (cnt / n_total)
        lane = jax.lax.broadcasted_iota(jnp.int32, (1, 16), 1)
        valid = (lane < _N_BINS) & (cnt > 0.0)
        ece = jnp.sum(jnp.where(valid, contrib, 0.0))
        out_ref[...] = jnp.full(out_ref.shape, ece, jnp.float32)


def kernel(logits, labels):
    n, c = logits.shape
    block_rows = 1000
    grid = n // block_rows
    labels3 = labels.astype(jnp.int32).reshape(grid, block_rows, 1)
    bounds = jnp.asarray(np.stack([_B_LO, _B_HI]))

    import functools
    out = pl.pallas_call(
        functools.partial(_ece_body, n_total=float(n)),
        grid=(grid,),
        in_specs=[
            pl.BlockSpec((block_rows, c), lambda i: (i, 0)),
            pl.BlockSpec((1, block_rows, 1), lambda i: (i, 0, 0)),
            pl.BlockSpec((2, 16), lambda i: (0, 0)),
        ],
        out_specs=pl.BlockSpec((8, 128), lambda i: (0, 0)),
        out_shape=jax.ShapeDtypeStruct((8, 128), jnp.float32),
        scratch_shapes=[pltpu.VMEM((8, 128), jnp.float32)],
        compiler_params=pltpu.CompilerParams(
            dimension_semantics=("arbitrary",),
        ),
    )(logits, labels3, bounds)
    return out[0, 0].reshape(1)


# speedup vs baseline: 1.0515x; 1.0515x over previous
import numpy as np
import jax
import jax.numpy as jnp
from jax.experimental import pallas as pl
from jax.experimental.pallas import tpu as pltpu

_N_BINS = 15
_B = np.linspace(0.0, 1.0, _N_BINS + 1)
_B_LO = np.zeros((16,), np.float32)
_B_HI = np.zeros((16,), np.float32)
_B_LO[:_N_BINS] = _B[:_N_BINS].astype(np.float32)
_B_HI[:_N_BINS] = _B[1:].astype(np.float32)
_B_LO[_N_BINS] = 2.0
_B_HI[_N_BINS] = 3.0


def _ece_body(*refs, n_total, nsplit):
    logits_refs = refs[:nsplit]
    labels_ref, bounds_ref, out_ref, acc_ref = refs[nsplit:]
    j = pl.program_id(0)
    nsteps = pl.num_programs(0)

    @pl.when(j == 0)
    def _init():
        acc_ref[...] = jnp.zeros_like(acc_ref)

    lo = bounds_ref[0:1, :]
    hi = bounds_ref[1:2, :]
    br = logits_refs[0].shape[0]

    for k in range(nsplit):
        x = logits_refs[k][...]
        m = jnp.max(x, axis=1, keepdims=True)
        s = jnp.sum(jnp.exp(x - m), axis=1, keepdims=True)
        conf = 1.0 / s

        col = jax.lax.broadcasted_iota(jnp.int32, x.shape, 1)
        amax = jnp.min(jnp.where(x == m, col, x.shape[1]), axis=1,
                       keepdims=True)
        lab = labels_ref[0, k * br:(k + 1) * br]
        acc = (amax == lab).astype(jnp.float32)

        inb = ((conf > lo) & (conf <= hi)).astype(jnp.float32)
        acc_ref[0:1, 0:16] += jnp.sum(inb, axis=0, keepdims=True)
        acc_ref[1:2, 0:16] += jnp.sum(inb * conf, axis=0, keepdims=True)
        acc_ref[2:3, 0:16] += jnp.sum(inb * acc, axis=0, keepdims=True)

    @pl.when(j == nsteps - 1)
    def _fini():
        cnt = acc_ref[0:1, 0:16]
        csum = acc_ref[1:2, 0:16]
        asum = acc_ref[2:3, 0:16]
        safe = jnp.maximum(cnt, 1.0)
        contrib = jnp.abs(csum / safe - asum / safe) * (cnt / n_total)
        lane = jax.lax.broadcasted_iota(jnp.int32, (1, 16), 1)
        valid = (lane < _N_BINS) & (cnt > 0.0)
        ece = jnp.sum(jnp.where(valid, contrib, 0.0))
        out_ref[...] = jnp.full(out_ref.shape, ece, jnp.float32)


def kernel(logits, labels):
    n, c = logits.shape
    nsplit = 4
    block_rows = 1000
    step_rows = nsplit * block_rows
    grid = n // step_rows
    labels3 = labels.astype(jnp.int32).reshape(grid, step_rows, 1)
    bounds = jnp.asarray(np.stack([_B_LO, _B_HI]))

    import functools
    logit_specs = [
        pl.BlockSpec((block_rows, c), lambda i, k=k: (i * nsplit + k, 0))
        for k in range(nsplit)
    ]
    out = pl.pallas_call(
        functools.partial(_ece_body, n_total=float(n), nsplit=nsplit),
        grid=(grid,),
        in_specs=logit_specs + [
            pl.BlockSpec((1, step_rows, 1), lambda i: (i, 0, 0)),
            pl.BlockSpec((2, 16), lambda i: (0, 0)),
        ],
        out_specs=pl.BlockSpec((8, 128), lambda i: (0, 0)),
        out_shape=jax.ShapeDtypeStruct((8, 128), jnp.float32),
        scratch_shapes=[pltpu.VMEM((8, 128), jnp.float32)],
        compiler_params=pltpu.CompilerParams(
            dimension_semantics=("arbitrary",),
        ),
    )(*([logits] * nsplit), labels3, bounds)
    return out[0, 0].reshape(1)
